# baseline (speedup 1.0000x reference)
import functools

import jax
import jax.numpy as jnp
import numpy as np
from jax import lax
from jax.experimental import pallas as pl
from jax.experimental.pallas import tpu as pltpu
from jax.experimental.pallas import tpu_sc as plsc

NC, NS, L = 2, 16, 16
NW = NC * NS
V, B, S, E, C = 1000000, 4096, 200, 64, 2
BPW = B // NW
GPB = BPW // L
TPC = 20
CHUNK = TPC * BPW
RING = 5
NCHUNK = S // TPC
NBLK = 65536
GRID = -(-V // NBLK)

_mesh = plsc.VectorSubcoreMesh(core_axis_name="c", subcore_axis_name="s")


def _project_body(w_ref, b_ref, tt_ref, o_ref):
    m = jnp.dot(w_ref[...], tt_ref[...], preferred_element_type=jnp.float32)
    m = m + b_ref[...][:, 0:1]
    u0 = lax.bitcast_convert_type(m[0].astype(jnp.bfloat16), jnp.uint16)
    u1 = lax.bitcast_convert_type(m[1].astype(jnp.bfloat16), jnp.uint16)
    o_ref[...] = (u0.astype(jnp.uint32)
                  | (u1.astype(jnp.uint32) << jnp.uint32(16)))


_tc_project = pl.pallas_call(
    _project_body,
    grid=(GRID,),
    in_specs=[
        pl.BlockSpec((8, E), lambda i: (0, 0)),
        pl.BlockSpec((8, 128), lambda i: (0, 0)),
        pl.BlockSpec((E, NBLK), lambda i: (0, i)),
    ],
    out_specs=pl.BlockSpec((NBLK,), lambda i: (i,)),
    out_shape=jax.ShapeDtypeStruct((V,), jnp.uint32),
)


@functools.partial(
    pl.kernel,
    out_type=jax.ShapeDtypeStruct((C, B), jnp.float32),
    mesh=_mesh,
    compiler_params=pltpu.CompilerParams(use_tc_tiling_on_sc=False,
                                         needs_layout_passes=False),
    scratch_types=(
        [pltpu.VMEM((NCHUNK, CHUNK), jnp.int32)]
        + [pltpu.VMEM((CHUNK,), jnp.uint32)
           for _ in range(NCHUNK)]
        + [pltpu.VMEM((C, BPW), jnp.float32),
           pltpu.VMEM_SHARED((V,), jnp.uint32),
           pltpu.SemaphoreType.DMA,
           pltpu.SemaphoreType.DMA]
    ),
)
def _sc_pool(pku, xr, out, idx_v, *rest):
    gbufs = rest[:NCHUNK]
    av, ptab = rest[NCHUNK], rest[NCHUNK + 1]
    sems = {0: rest[NCHUNK + 2], 1: rest[NCHUNK + 3]}
    c = lax.axis_index("c")
    s = lax.axis_index("s")

    pltpu.sync_copy(xr.at[c, s], idx_v)
    @pl.when(s < 8)
    def _():
        pltpu.sync_copy(pku.at[pl.ds(s * (V // 8), V // 8)],
                        ptab.at[pl.ds(s * (V // 8), V // 8)])
    plsc.subcore_barrier()

    def start(k, r, src, q):
        pltpu.async_copy(src.at[idx_v.at[k]], gbufs[r], sems[q])

    def wait(k, r, src, q):
        pltpu.make_async_copy(src.at[idx_v.at[k]], gbufs[r], sems[q]).wait()

    acc = [jnp.zeros((L,), jnp.float32) for _ in range(C * GPB)]

    def accumulate(r):
        for t in range(TPC):
            for g in range(GPB):
                v = gbufs[r][pl.ds(t * BPW + g * L, L)]
                a0, a1 = plsc.unpack(
                    plsc.bitcast(v, jnp.bfloat16),
                    format=plsc.PackFormat.INTERLEAVED)
                acc[g] = acc[g] + a0
                acc[GPB + g] = acc[GPB + g] + a1

    srcs = [ptab for k in range(NCHUNK)]
    qs = [1 for k in range(NCHUNK)]
    for k in range(NCHUNK):
        start(k, k, srcs[k], qs[k])
    for k in range(NCHUNK):
        wait(k, k, srcs[k], qs[k])
        accumulate(k)

    for cls in range(C):
        for g in range(GPB):
            av[cls, pl.ds(g * L, L)] = acc[cls * GPB + g]
    pltpu.sync_copy(av, out.at[:, pl.ds((c * NS + s) * BPW, BPW)])


def kernel(x, embed_table, fc_w, fc_b):
    tt = embed_table.T
    w8 = jnp.zeros((8, E), jnp.float32).at[:C].set(fc_w * (1.0 / S))
    b8 = jnp.zeros((8, 128), jnp.float32).at[:C, 0].set(fc_b * (1.0 / S))
    pku = _tc_project(w8, b8, tt)
    xr = (x.reshape(NC, NS, BPW, S).astype(jnp.int32)
          .transpose(0, 1, 3, 2).reshape(NC, NS, NCHUNK, CHUNK))
    out = _sc_pool(pku, xr)
    return out.T


# speedup vs baseline: 1.0074x; 1.0074x over previous
import functools

import jax
import jax.numpy as jnp
import numpy as np
from jax import lax
from jax.experimental import pallas as pl
from jax.experimental.pallas import tpu as pltpu
from jax.experimental.pallas import tpu_sc as plsc

NC, NS, L = 2, 16, 16
NW = NC * NS
V, B, S, E, C = 1000000, 4096, 200, 64, 2
BPW = B // NW
GPB = BPW // L
TPC = 20
CHUNK = TPC * BPW
RING = 5
NCHUNK = S // TPC
NBLK = 65536
GRID = -(-V // NBLK)

_mesh = plsc.VectorSubcoreMesh(core_axis_name="c", subcore_axis_name="s")


def _project_body(w_ref, b_ref, tt_ref, o_ref):
    m = jnp.dot(w_ref[...], tt_ref[...], preferred_element_type=jnp.float32)
    m = m + b_ref[...][:, 0:1]
    u0 = lax.bitcast_convert_type(m[0].astype(jnp.bfloat16), jnp.uint16)
    u1 = lax.bitcast_convert_type(m[1].astype(jnp.bfloat16), jnp.uint16)
    o_ref[...] = (u0.astype(jnp.uint32)
                  | (u1.astype(jnp.uint32) << jnp.uint32(16)))


_tc_project = pl.pallas_call(
    _project_body,
    grid=(GRID,),
    in_specs=[
        pl.BlockSpec((8, E), lambda i: (0, 0)),
        pl.BlockSpec((8, 128), lambda i: (0, 0)),
        pl.BlockSpec((E, NBLK), lambda i: (0, i)),
    ],
    out_specs=pl.BlockSpec((NBLK,), lambda i: (i,)),
    out_shape=jax.ShapeDtypeStruct((V,), jnp.uint32),
)


@functools.partial(
    pl.kernel,
    out_type=jax.ShapeDtypeStruct((C, B), jnp.float32),
    mesh=_mesh,
    compiler_params=pltpu.CompilerParams(use_tc_tiling_on_sc=False,
                                         needs_layout_passes=False),
    scratch_types=(
        [pltpu.VMEM((NCHUNK, CHUNK), jnp.int32)]
        + [pltpu.VMEM((CHUNK,), jnp.uint32)
           for _ in range(NCHUNK)]
        + [pltpu.VMEM((C, BPW), jnp.float32),
           pltpu.VMEM_SHARED((V,), jnp.uint32),
           pltpu.SemaphoreType.DMA,
           pltpu.SemaphoreType.DMA]
    ),
)
def _sc_pool(pku, xr, out, idx_v, *rest):
    gbufs = rest[:NCHUNK]
    av, ptab = rest[NCHUNK], rest[NCHUNK + 1]
    sems = {0: rest[NCHUNK + 2], 1: rest[NCHUNK + 3]}
    c = lax.axis_index("c")
    s = lax.axis_index("s")

    pltpu.sync_copy(xr.at[c, s], idx_v)
    @pl.when(s < 8)
    def _():
        pltpu.sync_copy(pku.at[pl.ds(s * (V // 8), V // 8)],
                        ptab.at[pl.ds(s * (V // 8), V // 8)])
    plsc.subcore_barrier()

    def start(k, r, src, q):
        pltpu.async_copy(src.at[idx_v.at[k]], gbufs[r], sems[q])

    def wait(k, r, src, q):
        pltpu.make_async_copy(src.at[idx_v.at[k]], gbufs[r], sems[q]).wait()

    acc = [jnp.zeros((L,), jnp.float32) for _ in range(C * GPB)]

    def accumulate(r):
        for t in range(TPC):
            for g in range(GPB):
                v = gbufs[r][pl.ds(t * BPW + g * L, L)]
                a0, a1 = plsc.unpack(
                    plsc.bitcast(v, jnp.bfloat16),
                    format=plsc.PackFormat.INTERLEAVED)
                acc[g] = acc[g] + a0
                acc[GPB + g] = acc[GPB + g] + a1

    srcs = [pku if k < 1 else ptab for k in range(NCHUNK)]
    qs = [0 if k < 1 else 1 for k in range(NCHUNK)]
    for k in range(NCHUNK):
        start(k, k, srcs[k], qs[k])
    for k in range(NCHUNK):
        wait(k, k, srcs[k], qs[k])
        accumulate(k)

    for cls in range(C):
        for g in range(GPB):
            av[cls, pl.ds(g * L, L)] = acc[cls * GPB + g]
    pltpu.sync_copy(av, out.at[:, pl.ds((c * NS + s) * BPW, BPW)])


def kernel(x, embed_table, fc_w, fc_b):
    tt = embed_table.T
    w8 = jnp.zeros((8, E), jnp.float32).at[:C].set(fc_w * (1.0 / S))
    b8 = jnp.zeros((8, 128), jnp.float32).at[:C, 0].set(fc_b * (1.0 / S))
    pku = _tc_project(w8, b8, tt)
    xr = (x.reshape(NC, NS, BPW, S).astype(jnp.int32)
          .transpose(0, 1, 3, 2).reshape(NC, NS, NCHUNK, CHUNK))
    out = _sc_pool(pku, xr)
    return out.T
